# baseline (speedup 1.0000x reference)
import functools
import jax
import jax.numpy as jnp
from jax.experimental import pallas as pl
from jax.experimental.pallas import tpu as pltpu

N = 4096
H = 128
ATT_H = 50
GAMMA = 0.1
NEG = -9e15

BR = 256
BC = 2048
RB = N // BR
CB = N // BC

PBR = 512


def _leaky(x):
    return jnp.where(x > 0, x, 0.2 * x)


def _elu(x):
    return jnp.where(x > 0, x, jnp.exp(jnp.minimum(x, 0.0)) - 1.0)


def _self_att2(z0, z1, Wp, bp, q):
    w0 = jnp.tanh(jnp.dot(z0, Wp, preferred_element_type=jnp.float32) + bp)
    w1 = jnp.tanh(jnp.dot(z1, Wp, preferred_element_type=jnp.float32) + bp)
    s0 = jnp.dot(w0, q, preferred_element_type=jnp.float32)
    s1 = jnp.dot(w1, q, preferred_element_type=jnp.float32)
    m = jnp.maximum(s0, s1)
    b0 = jnp.exp(s0 - m)
    b1 = jnp.exp(s1 - m)
    denom = b0 + b1
    return (b0 * z0 + b1 * z1) / denom


def _prologue_body(x0_ref, x1_ref, w10_ref, w11_ref,
                   a10_ref, a20_ref, a11_ref, a21_ref,
                   h0_ref, h1_ref, ev_ref):
    h0 = jnp.dot(x0_ref[...], w10_ref[...], preferred_element_type=jnp.float32)
    h1 = jnp.dot(x1_ref[...], w11_ref[...], preferred_element_type=jnp.float32)
    h0_ref[...] = h0.astype(jnp.bfloat16)
    h1_ref[...] = h1.astype(jnp.bfloat16)
    hs = (h0, h1)
    a1s = (a10_ref[...], a11_ref[...])
    a2s = (a20_ref[...], a21_ref[...])
    for t1 in range(2):
        for t2 in range(2):
            ev_ref[:, 2 * t1 + t2:2 * t1 + t2 + 1] = jnp.dot(
                hs[t1], a1s[t2], preferred_element_type=jnp.float32)
    for t in range(2):
        ev_ref[:, 4 + t:5 + t] = jnp.dot(
            hs[t], a2s[t], preferred_element_type=jnp.float32)
    ev_ref[:, 6:8] = jnp.zeros((PBR, 2), jnp.float32)


def _layer1_body(a00_ref, a01_ref, a10_ref, a11_ref,
                 h0_ref, h1_ref, ev_ref, evt_ref,
                 wp0_ref, bp0_ref, q0_ref, wp1_ref, bp1_ref, q1_ref,
                 w2_ref,
                 y0_ref, y1_ref,
                 ms_ref, pacc_ref, ajacc_ref):
    c = pl.program_id(1)
    adjs = (a00_ref[...], a01_ref[...], a10_ref[...], a11_ref[...])
    gs = (h0_ref[...], h1_ref[...])

    @pl.when(c == 0)
    def _init():
        ms_ref[:, 0:4] = jnp.full((BR, 4), NEG, jnp.float32)
        ms_ref[:, 4:8] = jnp.zeros((BR, 4), jnp.float32)
        pacc_ref[...] = jnp.zeros((BR, 4 * H), jnp.float32)
        ajacc_ref[...] = jnp.zeros((BR, 4 * H), jnp.float32)

    for p in range(4):
        t2 = p % 2
        a = adjs[p]
        e1 = ev_ref[:, p:p + 1]
        e2 = evt_ref[4 + t2:5 + t2, :]
        e = _leaky(e1 + e2)
        e = jnp.where(a > 0, e, NEG)
        m_old = ms_ref[:, p:p + 1]
        m_new = jnp.maximum(m_old, jnp.max(e, axis=1, keepdims=True))
        alpha = jnp.exp(m_old - m_new)
        pe = jnp.exp(e - m_new)
        ms_ref[:, p:p + 1] = m_new
        ms_ref[:, 4 + p:5 + p] = (ms_ref[:, 4 + p:5 + p] * alpha
                                  + jnp.sum(pe, axis=1, keepdims=True))
        g = gs[t2]
        sl = slice(p * H, (p + 1) * H)
        pacc_ref[:, sl] = (pacc_ref[:, sl] * alpha
                           + jnp.dot(pe.astype(jnp.bfloat16), g,
                                     preferred_element_type=jnp.float32))
        ajacc_ref[:, sl] = (ajacc_ref[:, sl]
                            + jnp.dot(a.astype(jnp.bfloat16), g,
                                      preferred_element_type=jnp.float32))

    @pl.when(c == CB - 1)
    def _fin():
        parts = []
        for p in range(4):
            sl = slice(p * H, (p + 1) * H)
            s = ms_ref[:, 4 + p:5 + p]
            parts.append(GAMMA * pacc_ref[:, sl] / s
                         + (1.0 - GAMMA) * ajacc_ref[:, sl])
        ats = ((wp0_ref[...], bp0_ref[...], q0_ref[...]),
               (wp1_ref[...], bp1_ref[...], q1_ref[...]))
        w2 = w2_ref[...]
        outs = (y0_ref, y1_ref)
        for t1 in range(2):
            xt = _self_att2(parts[2 * t1], parts[2 * t1 + 1], *ats[t1])
            xt = _elu(xt)
            outs[t1][...] = jnp.dot(
                xt, w2, preferred_element_type=jnp.float32
            ).astype(jnp.bfloat16)


def _layer2_body(a00_ref, a01_ref, a10_ref, a11_ref,
                 y0_ref, y1_ref, b2_ref,
                 wp0_ref, bp0_ref, q0_ref, wp1_ref, bp1_ref, q1_ref,
                 o0_ref, o1_ref,
                 acc_ref):
    c = pl.program_id(1)
    adjs = (a00_ref[...], a01_ref[...], a10_ref[...], a11_ref[...])
    ys = (y0_ref[...], y1_ref[...])

    @pl.when(c == 0)
    def _init():
        acc_ref[...] = jnp.zeros((BR, 4 * H), jnp.float32)

    for p in range(4):
        t2 = p % 2
        sl = slice(p * H, (p + 1) * H)
        acc_ref[:, sl] = acc_ref[:, sl] + jnp.dot(
            adjs[p].astype(jnp.bfloat16), ys[t2],
            preferred_element_type=jnp.float32)

    @pl.when(c == CB - 1)
    def _fin():
        b2 = b2_ref[...]
        parts = [acc_ref[:, p * H:(p + 1) * H] + b2 for p in range(4)]
        ats = ((wp0_ref[...], bp0_ref[...], q0_ref[...]),
               (wp1_ref[...], bp1_ref[...], q1_ref[...]))
        outs = (o0_ref, o1_ref)
        for t1 in range(2):
            xt = _self_att2(parts[2 * t1], parts[2 * t1 + 1], *ats[t1])
            outs[t1][...] = _elu(xt)


@jax.jit
def kernel(x0, x1, adj00, adj01, adj10, adj11,
           W1_0, W1_1, a1_0, a2_0, a1_1, a2_1,
           Wp1_0, bp1_0, q1_0, Wp1_1, bp1_1, q1_1,
           W2, b2, Wp2_0, bp2_0, q2_0, Wp2_1, bp2_1, q2_1):
    f32 = jnp.float32

    h0, h1, ev = pl.pallas_call(
        _prologue_body,
        grid=(N // PBR,),
        in_specs=[
            pl.BlockSpec((PBR, H), lambda r: (r, 0)),
            pl.BlockSpec((PBR, H), lambda r: (r, 0)),
            pl.BlockSpec((H, H), lambda r: (0, 0)),
            pl.BlockSpec((H, H), lambda r: (0, 0)),
            pl.BlockSpec((H, 1), lambda r: (0, 0)),
            pl.BlockSpec((H, 1), lambda r: (0, 0)),
            pl.BlockSpec((H, 1), lambda r: (0, 0)),
            pl.BlockSpec((H, 1), lambda r: (0, 0)),
        ],
        out_specs=[
            pl.BlockSpec((PBR, H), lambda r: (r, 0)),
            pl.BlockSpec((PBR, H), lambda r: (r, 0)),
            pl.BlockSpec((PBR, 8), lambda r: (r, 0)),
        ],
        out_shape=[
            jax.ShapeDtypeStruct((N, H), jnp.bfloat16),
            jax.ShapeDtypeStruct((N, H), jnp.bfloat16),
            jax.ShapeDtypeStruct((N, 8), f32),
        ],
    )(x0, x1, W1_0, W1_1, a1_0, a2_0, a1_1, a2_1)

    evt = ev.T

    bp1_0r = bp1_0.reshape(1, ATT_H)
    bp1_1r = bp1_1.reshape(1, ATT_H)
    bp2_0r = bp2_0.reshape(1, ATT_H)
    bp2_1r = bp2_1.reshape(1, ATT_H)
    b2r = b2.reshape(1, H)

    small = lambda shp: pl.BlockSpec(shp, lambda r, c: (0, 0))
    y0, y1 = pl.pallas_call(
        _layer1_body,
        grid=(RB, CB),
        in_specs=[
            pl.BlockSpec((BR, BC), lambda r, c: (r, c)),
            pl.BlockSpec((BR, BC), lambda r, c: (r, c)),
            pl.BlockSpec((BR, BC), lambda r, c: (r, c)),
            pl.BlockSpec((BR, BC), lambda r, c: (r, c)),
            pl.BlockSpec((BC, H), lambda r, c: (c, 0)),
            pl.BlockSpec((BC, H), lambda r, c: (c, 0)),
            pl.BlockSpec((BR, 8), lambda r, c: (r, 0)),
            pl.BlockSpec((8, BC), lambda r, c: (0, c)),
            small((H, ATT_H)), small((1, ATT_H)), small((ATT_H, 1)),
            small((H, ATT_H)), small((1, ATT_H)), small((ATT_H, 1)),
            small((H, H)),
        ],
        out_specs=[
            pl.BlockSpec((BR, H), lambda r, c: (r, 0)),
            pl.BlockSpec((BR, H), lambda r, c: (r, 0)),
        ],
        out_shape=[
            jax.ShapeDtypeStruct((N, H), jnp.bfloat16),
            jax.ShapeDtypeStruct((N, H), jnp.bfloat16),
        ],
        scratch_shapes=[
            pltpu.VMEM((BR, 8), f32),
            pltpu.VMEM((BR, 4 * H), f32),
            pltpu.VMEM((BR, 4 * H), f32),
        ],
    )(adj00, adj01, adj10, adj11, h0, h1, ev, evt,
      Wp1_0, bp1_0r, q1_0, Wp1_1, bp1_1r, q1_1, W2)

    o0, o1 = pl.pallas_call(
        _layer2_body,
        grid=(RB, CB),
        in_specs=[
            pl.BlockSpec((BR, BC), lambda r, c: (r, c)),
            pl.BlockSpec((BR, BC), lambda r, c: (r, c)),
            pl.BlockSpec((BR, BC), lambda r, c: (r, c)),
            pl.BlockSpec((BR, BC), lambda r, c: (r, c)),
            pl.BlockSpec((BC, H), lambda r, c: (c, 0)),
            pl.BlockSpec((BC, H), lambda r, c: (c, 0)),
            small((1, H)),
            small((H, ATT_H)), small((1, ATT_H)), small((ATT_H, 1)),
            small((H, ATT_H)), small((1, ATT_H)), small((ATT_H, 1)),
        ],
        out_specs=[
            pl.BlockSpec((BR, H), lambda r, c: (r, 0)),
            pl.BlockSpec((BR, H), lambda r, c: (r, 0)),
        ],
        out_shape=[
            jax.ShapeDtypeStruct((N, H), f32),
            jax.ShapeDtypeStruct((N, H), f32),
        ],
        scratch_shapes=[
            pltpu.VMEM((BR, 4 * H), f32),
        ],
    )(adj00, adj01, adj10, adj11, y0, y1, b2r,
      Wp2_0, bp2_0r, q2_0, Wp2_1, bp2_1r, q2_1)

    return (o0, o1)


# speedup vs baseline: 1.1691x; 1.1691x over previous
import functools
import jax
import jax.numpy as jnp
from jax.experimental import pallas as pl
from jax.experimental.pallas import tpu as pltpu

N = 4096
H = 128
ATT_H = 50
GAMMA = 0.1
NEG = -9e15

BR = 256
BC = 2048
RB = N // BR
CB = N // BC

PBR = 512


def _leaky(x):
    return jnp.where(x > 0, x, 0.2 * x)


def _elu(x):
    return jnp.where(x > 0, x, jnp.exp(jnp.minimum(x, 0.0)) - 1.0)


def _self_att2(z0, z1, Wp, bp, q):
    w0 = jnp.tanh(jnp.dot(z0, Wp, preferred_element_type=jnp.float32) + bp)
    w1 = jnp.tanh(jnp.dot(z1, Wp, preferred_element_type=jnp.float32) + bp)
    s0 = jnp.dot(w0, q, preferred_element_type=jnp.float32)
    s1 = jnp.dot(w1, q, preferred_element_type=jnp.float32)
    m = jnp.maximum(s0, s1)
    b0 = jnp.exp(s0 - m)
    b1 = jnp.exp(s1 - m)
    denom = b0 + b1
    return (b0 * z0 + b1 * z1) / denom


def _prologue_body(x0_ref, x1_ref, w10_ref, w11_ref,
                   a10_ref, a20_ref, a11_ref, a21_ref,
                   h0_ref, h1_ref, ev_ref):
    h0 = jnp.dot(x0_ref[...], w10_ref[...], preferred_element_type=jnp.float32)
    h1 = jnp.dot(x1_ref[...], w11_ref[...], preferred_element_type=jnp.float32)
    h0_ref[...] = h0
    h1_ref[...] = h1
    hs = (h0, h1)
    a1s = (a10_ref[...], a11_ref[...])
    a2s = (a20_ref[...], a21_ref[...])
    for t1 in range(2):
        for t2 in range(2):
            ev_ref[:, 2 * t1 + t2:2 * t1 + t2 + 1] = jnp.dot(
                hs[t1], a1s[t2], preferred_element_type=jnp.float32)
    for t in range(2):
        ev_ref[:, 4 + t:5 + t] = jnp.dot(
            hs[t], a2s[t], preferred_element_type=jnp.float32)
    ev_ref[:, 6:8] = jnp.zeros((PBR, 2), jnp.float32)


def _layer1_body(a00_ref, a01_ref, a10_ref, a11_ref,
                 h0_ref, h1_ref, ev_ref, evt_ref,
                 wp0_ref, bp0_ref, q0_ref, wp1_ref, bp1_ref, q1_ref,
                 w2_ref,
                 y0_ref, y1_ref,
                 ms_ref, pacc_ref, ajacc_ref):
    c = pl.program_id(1)
    adjs = (a00_ref[...], a01_ref[...], a10_ref[...], a11_ref[...])
    gs = (h0_ref[...], h1_ref[...])

    @pl.when(c == 0)
    def _init():
        ms_ref[:, 0:4] = jnp.full((BR, 4), NEG, jnp.float32)
        ms_ref[:, 4:8] = jnp.zeros((BR, 4), jnp.float32)
        pacc_ref[...] = jnp.zeros((BR, 4 * H), jnp.float32)
        ajacc_ref[...] = jnp.zeros((BR, 4 * H), jnp.float32)

    for p in range(4):
        t2 = p % 2
        a = adjs[p]
        e1 = ev_ref[:, p:p + 1]
        e2 = evt_ref[4 + t2:5 + t2, :]
        e = _leaky(e1 + e2)
        e = jnp.where(a > 0, e, NEG)
        m_old = ms_ref[:, p:p + 1]
        m_new = jnp.maximum(m_old, jnp.max(e, axis=1, keepdims=True))
        alpha = jnp.exp(m_old - m_new)
        pe = jnp.exp(e - m_new)
        ms_ref[:, p:p + 1] = m_new
        ms_ref[:, 4 + p:5 + p] = (ms_ref[:, 4 + p:5 + p] * alpha
                                  + jnp.sum(pe, axis=1, keepdims=True))
        g = gs[t2]
        sl = slice(p * H, (p + 1) * H)
        pacc_ref[:, sl] = (pacc_ref[:, sl] * alpha
                           + jnp.dot(pe, g, preferred_element_type=jnp.float32))
        ajacc_ref[:, sl] = (ajacc_ref[:, sl]
                            + jnp.dot(a, g, preferred_element_type=jnp.float32))

    @pl.when(c == CB - 1)
    def _fin():
        parts = []
        for p in range(4):
            sl = slice(p * H, (p + 1) * H)
            s = ms_ref[:, 4 + p:5 + p]
            parts.append(GAMMA * pacc_ref[:, sl] / s
                         + (1.0 - GAMMA) * ajacc_ref[:, sl])
        ats = ((wp0_ref[...], bp0_ref[...], q0_ref[...]),
               (wp1_ref[...], bp1_ref[...], q1_ref[...]))
        w2 = w2_ref[...]
        outs = (y0_ref, y1_ref)
        for t1 in range(2):
            xt = _self_att2(parts[2 * t1], parts[2 * t1 + 1], *ats[t1])
            xt = _elu(xt)
            outs[t1][...] = jnp.dot(xt, w2, preferred_element_type=jnp.float32)


def _layer2_body(a00_ref, a01_ref, a10_ref, a11_ref,
                 y0_ref, y1_ref, b2_ref,
                 wp0_ref, bp0_ref, q0_ref, wp1_ref, bp1_ref, q1_ref,
                 o0_ref, o1_ref,
                 acc_ref):
    c = pl.program_id(1)
    adjs = (a00_ref[...], a01_ref[...], a10_ref[...], a11_ref[...])
    ys = (y0_ref[...], y1_ref[...])

    @pl.when(c == 0)
    def _init():
        acc_ref[...] = jnp.zeros((BR, 4 * H), jnp.float32)

    for p in range(4):
        t2 = p % 2
        sl = slice(p * H, (p + 1) * H)
        acc_ref[:, sl] = acc_ref[:, sl] + jnp.dot(
            adjs[p], ys[t2], preferred_element_type=jnp.float32)

    @pl.when(c == CB - 1)
    def _fin():
        b2 = b2_ref[...]
        parts = [acc_ref[:, p * H:(p + 1) * H] + b2 for p in range(4)]
        ats = ((wp0_ref[...], bp0_ref[...], q0_ref[...]),
               (wp1_ref[...], bp1_ref[...], q1_ref[...]))
        outs = (o0_ref, o1_ref)
        for t1 in range(2):
            xt = _self_att2(parts[2 * t1], parts[2 * t1 + 1], *ats[t1])
            outs[t1][...] = _elu(xt)


@jax.jit
def kernel(x0, x1, adj00, adj01, adj10, adj11,
           W1_0, W1_1, a1_0, a2_0, a1_1, a2_1,
           Wp1_0, bp1_0, q1_0, Wp1_1, bp1_1, q1_1,
           W2, b2, Wp2_0, bp2_0, q2_0, Wp2_1, bp2_1, q2_1):
    f32 = jnp.float32

    h0, h1, ev = pl.pallas_call(
        _prologue_body,
        grid=(N // PBR,),
        in_specs=[
            pl.BlockSpec((PBR, H), lambda r: (r, 0)),
            pl.BlockSpec((PBR, H), lambda r: (r, 0)),
            pl.BlockSpec((H, H), lambda r: (0, 0)),
            pl.BlockSpec((H, H), lambda r: (0, 0)),
            pl.BlockSpec((H, 1), lambda r: (0, 0)),
            pl.BlockSpec((H, 1), lambda r: (0, 0)),
            pl.BlockSpec((H, 1), lambda r: (0, 0)),
            pl.BlockSpec((H, 1), lambda r: (0, 0)),
        ],
        out_specs=[
            pl.BlockSpec((PBR, H), lambda r: (r, 0)),
            pl.BlockSpec((PBR, H), lambda r: (r, 0)),
            pl.BlockSpec((PBR, 8), lambda r: (r, 0)),
        ],
        out_shape=[
            jax.ShapeDtypeStruct((N, H), f32),
            jax.ShapeDtypeStruct((N, H), f32),
            jax.ShapeDtypeStruct((N, 8), f32),
        ],
    )(x0, x1, W1_0, W1_1, a1_0, a2_0, a1_1, a2_1)

    evt = ev.T

    bp1_0r = bp1_0.reshape(1, ATT_H)
    bp1_1r = bp1_1.reshape(1, ATT_H)
    bp2_0r = bp2_0.reshape(1, ATT_H)
    bp2_1r = bp2_1.reshape(1, ATT_H)
    b2r = b2.reshape(1, H)

    small = lambda shp: pl.BlockSpec(shp, lambda r, c: (0, 0))
    y0, y1 = pl.pallas_call(
        _layer1_body,
        grid=(RB, CB),
        in_specs=[
            pl.BlockSpec((BR, BC), lambda r, c: (r, c)),
            pl.BlockSpec((BR, BC), lambda r, c: (r, c)),
            pl.BlockSpec((BR, BC), lambda r, c: (r, c)),
            pl.BlockSpec((BR, BC), lambda r, c: (r, c)),
            pl.BlockSpec((BC, H), lambda r, c: (c, 0)),
            pl.BlockSpec((BC, H), lambda r, c: (c, 0)),
            pl.BlockSpec((BR, 8), lambda r, c: (r, 0)),
            pl.BlockSpec((8, BC), lambda r, c: (0, c)),
            small((H, ATT_H)), small((1, ATT_H)), small((ATT_H, 1)),
            small((H, ATT_H)), small((1, ATT_H)), small((ATT_H, 1)),
            small((H, H)),
        ],
        out_specs=[
            pl.BlockSpec((BR, H), lambda r, c: (r, 0)),
            pl.BlockSpec((BR, H), lambda r, c: (r, 0)),
        ],
        out_shape=[
            jax.ShapeDtypeStruct((N, H), f32),
            jax.ShapeDtypeStruct((N, H), f32),
        ],
        scratch_shapes=[
            pltpu.VMEM((BR, 8), f32),
            pltpu.VMEM((BR, 4 * H), f32),
            pltpu.VMEM((BR, 4 * H), f32),
        ],
    )(adj00, adj01, adj10, adj11, h0, h1, ev, evt,
      Wp1_0, bp1_0r, q1_0, Wp1_1, bp1_1r, q1_1, W2)

    o0, o1 = pl.pallas_call(
        _layer2_body,
        grid=(RB, CB),
        in_specs=[
            pl.BlockSpec((BR, BC), lambda r, c: (r, c)),
            pl.BlockSpec((BR, BC), lambda r, c: (r, c)),
            pl.BlockSpec((BR, BC), lambda r, c: (r, c)),
            pl.BlockSpec((BR, BC), lambda r, c: (r, c)),
            pl.BlockSpec((BC, H), lambda r, c: (c, 0)),
            pl.BlockSpec((BC, H), lambda r, c: (c, 0)),
            small((1, H)),
            small((H, ATT_H)), small((1, ATT_H)), small((ATT_H, 1)),
            small((H, ATT_H)), small((1, ATT_H)), small((ATT_H, 1)),
        ],
        out_specs=[
            pl.BlockSpec((BR, H), lambda r, c: (r, 0)),
            pl.BlockSpec((BR, H), lambda r, c: (r, 0)),
        ],
        out_shape=[
            jax.ShapeDtypeStruct((N, H), f32),
            jax.ShapeDtypeStruct((N, H), f32),
        ],
        scratch_shapes=[
            pltpu.VMEM((BR, 4 * H), f32),
        ],
    )(adj00, adj01, adj10, adj11, y0, y1, b2r,
      Wp2_0, bp2_0r, q2_0, Wp2_1, bp2_1r, q2_1)

    return (o0, o1)


# speedup vs baseline: 1.6496x; 1.4110x over previous
import jax
import jax.numpy as jnp
from jax.experimental import pallas as pl

N = 4096
H = 128
ATT_H = 50
GAMMA = 0.1
NEG = -9e15

BR = 128
RB = N // BR

PBR = 512


def _leaky(x):
    return jnp.maximum(x, 0.2 * x)


def _elu(x):
    return jnp.where(x > 0, x, jnp.exp(jnp.minimum(x, 0.0)) - 1.0)


def _self_att2(z0, z1, Wp, bp, q):
    w0 = jnp.tanh(jnp.dot(z0, Wp, preferred_element_type=jnp.float32) + bp)
    w1 = jnp.tanh(jnp.dot(z1, Wp, preferred_element_type=jnp.float32) + bp)
    s0 = jnp.dot(w0, q, preferred_element_type=jnp.float32)
    s1 = jnp.dot(w1, q, preferred_element_type=jnp.float32)
    m = jnp.maximum(s0, s1)
    b0 = jnp.exp(s0 - m)
    b1 = jnp.exp(s1 - m)
    denom = b0 + b1
    return (b0 * z0 + b1 * z1) / denom


def _prologue_body(x0_ref, x1_ref, w10_ref, w11_ref,
                   a10_ref, a20_ref, a11_ref, a21_ref,
                   h0_ref, h1_ref, ev_ref):
    h0 = jnp.dot(x0_ref[...], w10_ref[...], preferred_element_type=jnp.float32)
    h1 = jnp.dot(x1_ref[...], w11_ref[...], preferred_element_type=jnp.float32)
    h0_ref[...] = h0
    h1_ref[...] = h1
    hs = (h0, h1)
    a1s = (a10_ref[...], a11_ref[...])
    a2s = (a20_ref[...], a21_ref[...])
    for t1 in range(2):
        for t2 in range(2):
            ev_ref[:, 2 * t1 + t2:2 * t1 + t2 + 1] = jnp.dot(
                hs[t1], a1s[t2], preferred_element_type=jnp.float32)
    for t in range(2):
        ev_ref[:, 4 + t:5 + t] = jnp.dot(
            hs[t], a2s[t], preferred_element_type=jnp.float32)
    ev_ref[:, 6:8] = jnp.zeros((PBR, 2), jnp.float32)


def _layer1_body(a00_ref, a01_ref, a10_ref, a11_ref,
                 h0_ref, h1_ref, ev_ref, evt_ref,
                 wp0_ref, bp0_ref, q0_ref, wp1_ref, bp1_ref, q1_ref,
                 w2_ref,
                 y0_ref, y1_ref):
    adj_refs = (a00_ref, a01_ref, a10_ref, a11_ref)
    parts = [None] * 4
    for t2 in range(2):
        g = (h0_ref, h1_ref)[t2][...]
        e2 = evt_ref[4 + t2:5 + t2, :]
        ops = []
        sums = []
        for t1 in range(2):
            p = 2 * t1 + t2
            a = adj_refs[p][...]
            e1 = ev_ref[:, p:p + 1]
            e = _leaky(e1 + e2)
            e = jnp.where(a > 0, e, NEG)
            m = jnp.max(e, axis=1, keepdims=True)
            pe = jnp.exp(e - m)
            sums.append(jnp.sum(pe, axis=1, keepdims=True))
            ops.append(pe)
            ops.append(a)
        res = jnp.dot(jnp.concatenate(ops, axis=0), g,
                      preferred_element_type=jnp.float32)
        for t1 in range(2):
            p = 2 * t1 + t2
            pg = res[2 * t1 * BR:(2 * t1 + 1) * BR, :]
            ag = res[(2 * t1 + 1) * BR:(2 * t1 + 2) * BR, :]
            parts[p] = GAMMA * pg / sums[t1] + (1.0 - GAMMA) * ag
    ats = ((wp0_ref[...], bp0_ref[...], q0_ref[...]),
           (wp1_ref[...], bp1_ref[...], q1_ref[...]))
    w2 = w2_ref[...]
    outs = (y0_ref, y1_ref)
    for t1 in range(2):
        xt = _self_att2(parts[2 * t1], parts[2 * t1 + 1], *ats[t1])
        xt = _elu(xt)
        outs[t1][...] = jnp.dot(xt, w2, preferred_element_type=jnp.float32)


def _layer2_body(a00_ref, a01_ref, a10_ref, a11_ref,
                 y0_ref, y1_ref, b2_ref,
                 wp0_ref, bp0_ref, q0_ref, wp1_ref, bp1_ref, q1_ref,
                 o0_ref, o1_ref):
    adj_refs = (a00_ref, a01_ref, a10_ref, a11_ref)
    b2 = b2_ref[...]
    parts = [None] * 4
    for t2 in range(2):
        y = (y0_ref, y1_ref)[t2][...]
        stacked = jnp.concatenate(
            [adj_refs[t2][...], adj_refs[2 + t2][...]], axis=0)
        res = jnp.dot(stacked, y, preferred_element_type=jnp.float32)
        parts[t2] = res[:BR, :] + b2
        parts[2 + t2] = res[BR:, :] + b2
    ats = ((wp0_ref[...], bp0_ref[...], q0_ref[...]),
           (wp1_ref[...], bp1_ref[...], q1_ref[...]))
    outs = (o0_ref, o1_ref)
    for t1 in range(2):
        xt = _self_att2(parts[2 * t1], parts[2 * t1 + 1], *ats[t1])
        outs[t1][...] = _elu(xt)


@jax.jit
def kernel(x0, x1, adj00, adj01, adj10, adj11,
           W1_0, W1_1, a1_0, a2_0, a1_1, a2_1,
           Wp1_0, bp1_0, q1_0, Wp1_1, bp1_1, q1_1,
           W2, b2, Wp2_0, bp2_0, q2_0, Wp2_1, bp2_1, q2_1):
    f32 = jnp.float32

    h0, h1, ev = pl.pallas_call(
        _prologue_body,
        grid=(N // PBR,),
        in_specs=[
            pl.BlockSpec((PBR, H), lambda r: (r, 0)),
            pl.BlockSpec((PBR, H), lambda r: (r, 0)),
            pl.BlockSpec((H, H), lambda r: (0, 0)),
            pl.BlockSpec((H, H), lambda r: (0, 0)),
            pl.BlockSpec((H, 1), lambda r: (0, 0)),
            pl.BlockSpec((H, 1), lambda r: (0, 0)),
            pl.BlockSpec((H, 1), lambda r: (0, 0)),
            pl.BlockSpec((H, 1), lambda r: (0, 0)),
        ],
        out_specs=[
            pl.BlockSpec((PBR, H), lambda r: (r, 0)),
            pl.BlockSpec((PBR, H), lambda r: (r, 0)),
            pl.BlockSpec((PBR, 8), lambda r: (r, 0)),
        ],
        out_shape=[
            jax.ShapeDtypeStruct((N, H), f32),
            jax.ShapeDtypeStruct((N, H), f32),
            jax.ShapeDtypeStruct((N, 8), f32),
        ],
    )(x0, x1, W1_0, W1_1, a1_0, a2_0, a1_1, a2_1)

    evt = ev.T

    bp1_0r = bp1_0.reshape(1, ATT_H)
    bp1_1r = bp1_1.reshape(1, ATT_H)
    bp2_0r = bp2_0.reshape(1, ATT_H)
    bp2_1r = bp2_1.reshape(1, ATT_H)
    b2r = b2.reshape(1, H)

    rowspec = pl.BlockSpec((BR, N), lambda r: (r, 0))
    outspec = pl.BlockSpec((BR, H), lambda r: (r, 0))
    full = lambda shp: pl.BlockSpec(shp, lambda r: (0, 0))

    y0, y1 = pl.pallas_call(
        _layer1_body,
        grid=(RB,),
        in_specs=[
            rowspec, rowspec, rowspec, rowspec,
            full((N, H)), full((N, H)),
            pl.BlockSpec((BR, 8), lambda r: (r, 0)),
            full((8, N)),
            full((H, ATT_H)), full((1, ATT_H)), full((ATT_H, 1)),
            full((H, ATT_H)), full((1, ATT_H)), full((ATT_H, 1)),
            full((H, H)),
        ],
        out_specs=[outspec, outspec],
        out_shape=[
            jax.ShapeDtypeStruct((N, H), f32),
            jax.ShapeDtypeStruct((N, H), f32),
        ],
    )(adj00, adj01, adj10, adj11, h0, h1, ev, evt,
      Wp1_0, bp1_0r, q1_0, Wp1_1, bp1_1r, q1_1, W2)

    o0, o1 = pl.pallas_call(
        _layer2_body,
        grid=(RB,),
        in_specs=[
            rowspec, rowspec, rowspec, rowspec,
            full((N, H)), full((N, H)),
            full((1, H)),
            full((H, ATT_H)), full((1, ATT_H)), full((ATT_H, 1)),
            full((H, ATT_H)), full((1, ATT_H)), full((ATT_H, 1)),
        ],
        out_specs=[outspec, outspec],
        out_shape=[
            jax.ShapeDtypeStruct((N, H), f32),
            jax.ShapeDtypeStruct((N, H), f32),
        ],
    )(adj00, adj01, adj10, adj11, y0, y1, b2r,
      Wp2_0, bp2_0r, q2_0, Wp2_1, bp2_1r, q2_1)

    return (o0, o1)


# speedup vs baseline: 2.1613x; 1.3102x over previous
import jax
import jax.numpy as jnp
from jax.experimental import pallas as pl

N = 4096
H = 128
HA = H + 8
ATT_H = 50
GAMMA = 0.1

BR = 128
RB = N // BR

PBR = 512


def _leaky(x):
    return jnp.maximum(x, 0.2 * x)


def _elu(x):
    return jnp.where(x > 0, x, jnp.exp(jnp.minimum(x, 0.0)) - 1.0)


def _self_att2(z0, z1, Wp, bp, q):
    w0 = jnp.tanh(jnp.dot(z0, Wp, preferred_element_type=jnp.float32) + bp)
    w1 = jnp.tanh(jnp.dot(z1, Wp, preferred_element_type=jnp.float32) + bp)
    s0 = jnp.dot(w0, q, preferred_element_type=jnp.float32)
    s1 = jnp.dot(w1, q, preferred_element_type=jnp.float32)
    m = jnp.maximum(s0, s1)
    b0 = jnp.exp(s0 - m)
    b1 = jnp.exp(s1 - m)
    denom = b0 + b1
    return (b0 * z0 + b1 * z1) / denom


def _prologue_body(x0_ref, x1_ref, w10_ref, w11_ref,
                   a10_ref, a20_ref, a11_ref, a21_ref,
                   h0_ref, h1_ref, ev_ref, hm_ref):
    r = pl.program_id(0)
    h0 = jnp.dot(x0_ref[...], w10_ref[...], preferred_element_type=jnp.float32)
    h1 = jnp.dot(x1_ref[...], w11_ref[...], preferred_element_type=jnp.float32)
    h0_ref[:, :H] = h0
    h1_ref[:, :H] = h1
    h0_ref[:, H:] = jnp.ones((PBR, 8), jnp.float32)
    h1_ref[:, H:] = jnp.ones((PBR, 8), jnp.float32)

    @pl.when(r == 0)
    def _init():
        hm_ref[...] = jnp.zeros((8, H), jnp.float32)

    hm_ref[0:1, :] += jnp.sum(h0, axis=0, keepdims=True)
    hm_ref[1:2, :] += jnp.sum(h1, axis=0, keepdims=True)

    hs = (h0, h1)
    a1s = (a10_ref[...], a11_ref[...])
    a2s = (a20_ref[...], a21_ref[...])
    for t1 in range(2):
        for t2 in range(2):
            ev_ref[:, 2 * t1 + t2:2 * t1 + t2 + 1] = jnp.dot(
                hs[t1], a1s[t2], preferred_element_type=jnp.float32)
    for t in range(2):
        ev_ref[:, 4 + t:5 + t] = jnp.dot(
            hs[t], a2s[t], preferred_element_type=jnp.float32)
    ev_ref[:, 6:8] = jnp.zeros((PBR, 2), jnp.float32)


def _layer1_body(a00_ref, a01_ref, a10_ref, a11_ref,
                 h0_ref, h1_ref, ev_ref, evt_ref, hm_ref,
                 wp0_ref, bp0_ref, q0_ref, wp1_ref, bp1_ref, q1_ref,
                 w2_ref,
                 y0_ref, y1_ref):
    adj_refs = (a00_ref, a01_ref, a10_ref, a11_ref)
    parts = [None] * 4
    for t2 in range(2):
        g = (h0_ref, h1_ref)[t2][...]
        e2 = evt_ref[4 + t2:5 + t2, :]
        hmean = hm_ref[t2:t2 + 1, :] * (1.0 / N)
        ops = []
        for t1 in range(2):
            p = 2 * t1 + t2
            a = adj_refs[p][...]
            e1 = ev_ref[:, p:p + 1]
            pe = jnp.where(a > 0, jnp.exp(_leaky(e1 + e2)), 0.0)
            ops.append(pe)
            ops.append(a)
        res = jnp.dot(jnp.concatenate(ops, axis=0), g,
                      preferred_element_type=jnp.float32)
        for t1 in range(2):
            p = 2 * t1 + t2
            pg = res[2 * t1 * BR:(2 * t1 + 1) * BR, :H]
            s = res[2 * t1 * BR:(2 * t1 + 1) * BR, H:H + 1]
            ag = res[(2 * t1 + 1) * BR:(2 * t1 + 2) * BR, :H]
            empty = s <= 0.0
            soft = jnp.where(empty, hmean, pg / jnp.where(empty, 1.0, s))
            parts[p] = GAMMA * soft + (1.0 - GAMMA) * ag
    ats = ((wp0_ref[...], bp0_ref[...], q0_ref[...]),
           (wp1_ref[...], bp1_ref[...], q1_ref[...]))
    w2 = w2_ref[...]
    outs = (y0_ref, y1_ref)
    for t1 in range(2):
        xt = _self_att2(parts[2 * t1], parts[2 * t1 + 1], *ats[t1])
        xt = _elu(xt)
        outs[t1][...] = jnp.dot(xt, w2, preferred_element_type=jnp.float32)


def _layer2_body(a00_ref, a01_ref, a10_ref, a11_ref,
                 y0_ref, y1_ref, b2_ref,
                 wp0_ref, bp0_ref, q0_ref, wp1_ref, bp1_ref, q1_ref,
                 o0_ref, o1_ref):
    adj_refs = (a00_ref, a01_ref, a10_ref, a11_ref)
    b2 = b2_ref[...]
    parts = [None] * 4
    for t2 in range(2):
        y = (y0_ref, y1_ref)[t2][...]
        stacked = jnp.concatenate(
            [adj_refs[t2][...], adj_refs[2 + t2][...]], axis=0)
        res = jnp.dot(stacked, y, preferred_element_type=jnp.float32)
        parts[t2] = res[:BR, :] + b2
        parts[2 + t2] = res[BR:, :] + b2
    ats = ((wp0_ref[...], bp0_ref[...], q0_ref[...]),
           (wp1_ref[...], bp1_ref[...], q1_ref[...]))
    outs = (o0_ref, o1_ref)
    for t1 in range(2):
        xt = _self_att2(parts[2 * t1], parts[2 * t1 + 1], *ats[t1])
        outs[t1][...] = _elu(xt)


@jax.jit
def kernel(x0, x1, adj00, adj01, adj10, adj11,
           W1_0, W1_1, a1_0, a2_0, a1_1, a2_1,
           Wp1_0, bp1_0, q1_0, Wp1_1, bp1_1, q1_1,
           W2, b2, Wp2_0, bp2_0, q2_0, Wp2_1, bp2_1, q2_1):
    f32 = jnp.float32

    h0, h1, ev, hm = pl.pallas_call(
        _prologue_body,
        grid=(N // PBR,),
        in_specs=[
            pl.BlockSpec((PBR, H), lambda r: (r, 0)),
            pl.BlockSpec((PBR, H), lambda r: (r, 0)),
            pl.BlockSpec((H, H), lambda r: (0, 0)),
            pl.BlockSpec((H, H), lambda r: (0, 0)),
            pl.BlockSpec((H, 1), lambda r: (0, 0)),
            pl.BlockSpec((H, 1), lambda r: (0, 0)),
            pl.BlockSpec((H, 1), lambda r: (0, 0)),
            pl.BlockSpec((H, 1), lambda r: (0, 0)),
        ],
        out_specs=[
            pl.BlockSpec((PBR, HA), lambda r: (r, 0)),
            pl.BlockSpec((PBR, HA), lambda r: (r, 0)),
            pl.BlockSpec((PBR, 8), lambda r: (r, 0)),
            pl.BlockSpec((8, H), lambda r: (0, 0)),
        ],
        out_shape=[
            jax.ShapeDtypeStruct((N, HA), f32),
            jax.ShapeDtypeStruct((N, HA), f32),
            jax.ShapeDtypeStruct((N, 8), f32),
            jax.ShapeDtypeStruct((8, H), f32),
        ],
    )(x0, x1, W1_0, W1_1, a1_0, a2_0, a1_1, a2_1)

    evt = ev.T

    bp1_0r = bp1_0.reshape(1, ATT_H)
    bp1_1r = bp1_1.reshape(1, ATT_H)
    bp2_0r = bp2_0.reshape(1, ATT_H)
    bp2_1r = bp2_1.reshape(1, ATT_H)
    b2r = b2.reshape(1, H)

    rowspec = pl.BlockSpec((BR, N), lambda r: (r, 0))
    outspec = pl.BlockSpec((BR, H), lambda r: (r, 0))
    full = lambda shp: pl.BlockSpec(shp, lambda r: (0, 0))

    y0, y1 = pl.pallas_call(
        _layer1_body,
        grid=(RB,),
        in_specs=[
            rowspec, rowspec, rowspec, rowspec,
            full((N, HA)), full((N, HA)),
            pl.BlockSpec((BR, 8), lambda r: (r, 0)),
            full((8, N)),
            full((8, H)),
            full((H, ATT_H)), full((1, ATT_H)), full((ATT_H, 1)),
            full((H, ATT_H)), full((1, ATT_H)), full((ATT_H, 1)),
            full((H, H)),
        ],
        out_specs=[outspec, outspec],
        out_shape=[
            jax.ShapeDtypeStruct((N, H), f32),
            jax.ShapeDtypeStruct((N, H), f32),
        ],
    )(adj00, adj01, adj10, adj11, h0, h1, ev, evt, hm,
      Wp1_0, bp1_0r, q1_0, Wp1_1, bp1_1r, q1_1, W2)

    o0, o1 = pl.pallas_call(
        _layer2_body,
        grid=(RB,),
        in_specs=[
            rowspec, rowspec, rowspec, rowspec,
            full((N, H)), full((N, H)),
            full((1, H)),
            full((H, ATT_H)), full((1, ATT_H)), full((ATT_H, 1)),
            full((H, ATT_H)), full((1, ATT_H)), full((ATT_H, 1)),
        ],
        out_specs=[outspec, outspec],
        out_shape=[
            jax.ShapeDtypeStruct((N, H), f32),
            jax.ShapeDtypeStruct((N, H), f32),
        ],
    )(adj00, adj01, adj10, adj11, y0, y1, b2r,
      Wp2_0, bp2_0r, q2_0, Wp2_1, bp2_1r, q2_1)

    return (o0, o1)
